# Initial kernel scaffold; baseline (speedup 1.0000x reference)
import jax
import jax.numpy as jnp
from jax.experimental import pallas as pl


def kernel(x, embeddings, cluster_size, unnormalized):
    raise NotImplementedError("write your pallas kernel here")


# speedup vs baseline: 2.1459x; 2.1459x over previous
import functools

import jax
import jax.numpy as jnp
from jax import lax
from jax.experimental import pallas as pl
from jax.experimental.pallas import tpu as pltpu

EMBED_DIM = 64
NUM_EMB = 1024
COMMIT = 0.25
MOMENTUM = 0.9

ROWS = 32 * 1024
BLK = 2048
NBLK = ROWS // BLK


def _body(x_ref, e_ref, cs_ref, un_ref,
          q_ref, idx_ref, loss_ref, ppl_ref, newe_ref, newcs_ref, newun_ref,
          cs_acc, un_acc, loss_acc):
    i = pl.program_id(0)

    xb = x_ref[...]
    e = e_ref[...]

    x2 = jnp.sum(xb * xb, axis=1, keepdims=True)
    e2 = jnp.sum(e * e, axis=0, keepdims=True)
    s = lax.dot_general(xb, e, (((1,), (0,)), ((), ())),
                        preferred_element_type=jnp.float32)
    dist = (x2 + e2) - 2.0 * s
    idx = jnp.argmin(dist, axis=1).astype(jnp.int32)
    idx_ref[...] = idx

    codes = lax.broadcasted_iota(jnp.int32, (BLK, NUM_EMB), 1)
    enc = (codes == idx[:, None]).astype(jnp.float32)

    q = lax.dot_general(enc, e, (((1,), (1,)), ((), ())),
                        preferred_element_type=jnp.float32)
    q_ref[...] = xb + (q - xb)

    cs_part = jnp.sum(enc, axis=0)
    un_part = lax.dot_general(xb, enc, (((0,), (0,)), ((), ())),
                              preferred_element_type=jnp.float32)
    loss_part = jnp.sum((q - xb) ** 2).reshape(1, 1)

    @pl.when(i == 0)
    def _init():
        cs_acc[...] = cs_part
        un_acc[...] = un_part
        loss_acc[...] = loss_part

    @pl.when(i > 0)
    def _acc():
        cs_acc[...] += cs_part
        un_acc[...] += un_part
        loss_acc[...] += loss_part

    @pl.when(i == NBLK - 1)
    def _finalize():
        counts = cs_acc[...]
        new_cs = (1.0 - MOMENTUM) * counts + MOMENTUM * cs_ref[...]
        n = jnp.sum(new_cs)
        stable_cs = (new_cs + 1e-20) / (n + NUM_EMB * 1e-20) * n
        new_un = (1.0 - MOMENTUM) * un_acc[...] + MOMENTUM * un_ref[...]
        newcs_ref[...] = new_cs
        newun_ref[...] = new_un
        newe_ref[...] = new_un / stable_cs[None, :]
        probs = counts * (1.0 / ROWS)
        ppl_ref[...] = jnp.exp(
            -jnp.sum(probs * jnp.log(probs + 1e-20))).reshape(1, 1)
        loss_ref[...] = COMMIT * loss_acc[...] * (1.0 / (ROWS * EMBED_DIM))


def kernel(x, embeddings, cluster_size, unnormalized):
    input_shape = x.shape[:-1]
    xf = x.reshape((-1, EMBED_DIM))

    grid = (NBLK,)
    out_shapes = (
        jax.ShapeDtypeStruct((ROWS, EMBED_DIM), jnp.float32),
        jax.ShapeDtypeStruct((ROWS,), jnp.int32),
        jax.ShapeDtypeStruct((1, 1), jnp.float32),
        jax.ShapeDtypeStruct((1, 1), jnp.float32),
        jax.ShapeDtypeStruct((EMBED_DIM, NUM_EMB), jnp.float32),
        jax.ShapeDtypeStruct((NUM_EMB,), jnp.float32),
        jax.ShapeDtypeStruct((EMBED_DIM, NUM_EMB), jnp.float32),
    )
    full2d = pl.BlockSpec((EMBED_DIM, NUM_EMB), lambda i: (0, 0))
    full1d = pl.BlockSpec((NUM_EMB,), lambda i: (0,))
    scalar = pl.BlockSpec((1, 1), lambda i: (0, 0))

    q, idx, loss, ppl, new_e, new_cs, new_un = pl.pallas_call(
        _body,
        grid=grid,
        in_specs=[
            pl.BlockSpec((BLK, EMBED_DIM), lambda i: (i, 0)),
            full2d,
            full1d,
            full2d,
        ],
        out_specs=(
            pl.BlockSpec((BLK, EMBED_DIM), lambda i: (i, 0)),
            pl.BlockSpec((BLK,), lambda i: (i,)),
            scalar,
            scalar,
            full2d,
            full1d,
            full2d,
        ),
        out_shape=out_shapes,
        scratch_shapes=[
            pltpu.VMEM((NUM_EMB,), jnp.float32),
            pltpu.VMEM((EMBED_DIM, NUM_EMB), jnp.float32),
            pltpu.VMEM((1, 1), jnp.float32),
        ],
    )(xf, embeddings, cluster_size, unnormalized)

    quantized_st = q.reshape((*input_shape, EMBED_DIM))
    return (quantized_st, loss.reshape(()), ppl.reshape(()), idx,
            new_e, new_cs, new_un)
